# Initial kernel scaffold; baseline (speedup 1.0000x reference)
import jax
import jax.numpy as jnp
from jax.experimental import pallas as pl


def kernel(x1, x2, edge_index1, edge_index2, n1_W1, n1_b1, n1_rW1, n1_rb1, n1_g1, n1_bt1, n1_W2, n1_b2, n1_rW2, n1_rb2, n1_g2, n1_bt2, n1_aw, n1_ab, n1_pW1, n1_pb1, n1_pg, n1_pbt, n1_pW2, n1_pb2, n2_W1, n2_b1, n2_rW1, n2_rb1, n2_g1, n2_bt1, n2_W2, n2_b2, n2_rW2, n2_rb2, n2_g2, n2_bt2, n2_aw, n2_ab, n2_pW1, n2_pb1, n2_pg, n2_pbt, n2_pW2, n2_pb2, predW, predb):
    raise NotImplementedError("write your pallas kernel here")


# speedup vs baseline: 3.1867x; 3.1867x over previous
import functools

import jax
import jax.numpy as jnp
from jax import lax
from jax.experimental import pallas as pl
from jax.experimental.pallas import tpu as pltpu
from jax.experimental.pallas import tpu_sc as plsc

N = 10000
E = 320000
D = 128
H = 32

NC = 2
NS = 16
CH = 128
EPT = E // NS
EPT_PAD = 20480
NCHUNK = EPT_PAD // CH
DUMP = 2 * N
TBL_R = 20224
ZCH = TBL_R // NS
CPS = 624
CPW = 640
DW = 8
NQ = 2 * NS * NCHUNK

_f32 = jnp.float32

def _mesh():
    return plsc.VectorSubcoreMesh(
        core_axis_name="c", subcore_axis_name="s", num_cores=NC, num_subcores=NS)


@functools.lru_cache(maxsize=None)
def _build_sc_degrees():
  @functools.partial(
      pl.kernel,
      out_type=[jax.ShapeDtypeStruct((2 * N, DW), _f32),
                jax.ShapeDtypeStruct((2 * N, DW), _f32)],
      mesh=_mesh(),
      scratch_types=[
          pltpu.VMEM((1, CH), jnp.int32),
          pltpu.VMEM((1, CH), jnp.int32),
          pltpu.VMEM((CH, DW), _f32),
          pltpu.VMEM_SHARED((TBL_R, DW), _f32),
          pltpu.VMEM_SHARED((TBL_R, DW), _f32),
      ],
      compiler_params=pltpu.CompilerParams(use_tc_tiling_on_sc=False),
  )
  def deg(src_hbm, dst_hbm, ones_hbm, zeros_hbm, od_hbm, id_hbm,
          sidx, didx, ones_v, od_sh, id_sh):
    c = lax.axis_index("c")
    s = lax.axis_index("s")
    w = c * NS + s
    pltpu.sync_copy(zeros_hbm.at[pl.ds(s * ZCH, ZCH)], od_sh.at[pl.ds(s * ZCH, ZCH)])
    pltpu.sync_copy(zeros_hbm.at[pl.ds(s * ZCH, ZCH)], id_sh.at[pl.ds(s * ZCH, ZCH)])
    pltpu.sync_copy(ones_hbm, ones_v)
    plsc.subcore_barrier()

    def body(j, carry):
      q = w * NCHUNK + j
      pltpu.sync_copy(src_hbm.at[q], sidx)
      pltpu.sync_copy(ones_v, od_sh.at[sidx.at[0]], add=True)
      pltpu.sync_copy(dst_hbm.at[q], didx)
      pltpu.sync_copy(ones_v, id_sh.at[didx.at[0]], add=True)
      return carry

    lax.fori_loop(0, NCHUNK, body, 0)
    plsc.subcore_barrier()
    base = c * N + s * CPS
    pltpu.sync_copy(od_sh.at[pl.ds(base, CPW)], od_hbm.at[pl.ds(base, CPW)])
    pltpu.sync_copy(id_sh.at[pl.ds(base, CPW)], id_hbm.at[pl.ds(base, CPW)])

  return deg


def _sc_degrees(*args):
    return _build_sc_degrees()(*args)


@functools.lru_cache(maxsize=None)
def _build_sc_edge_agg():
  @functools.partial(
      pl.kernel,
      out_type=jax.ShapeDtypeStruct((2 * N, H), _f32),
      mesh=_mesh(),
      scratch_types=[
          pltpu.VMEM((1, CH), jnp.int32),
          pltpu.VMEM((1, CH), jnp.int32),
          pltpu.VMEM((CH, H), _f32),
          pltpu.VMEM_SHARED((TBL_R, H), _f32),
          pltpu.SemaphoreType.DMA,
      ],
      compiler_params=pltpu.CompilerParams(use_tc_tiling_on_sc=False),
  )
  def agg(src_hbm, dst_hbm, table_hbm, zeros_hbm, agg_hbm,
          sidx, didx, rows_v, agg_sh, sem):
    c = lax.axis_index("c")
    s = lax.axis_index("s")
    w = c * NS + s
    pltpu.sync_copy(zeros_hbm.at[pl.ds(s * ZCH, ZCH)], agg_sh.at[pl.ds(s * ZCH, ZCH)])
    plsc.subcore_barrier()

    def body(j, carry):
      q = w * NCHUNK + j
      pltpu.sync_copy(src_hbm.at[q], sidx)
      pltpu.async_copy(table_hbm.at[sidx.at[0]], rows_v, sem).wait()
      pltpu.sync_copy(dst_hbm.at[q], didx)
      pltpu.sync_copy(rows_v, agg_sh.at[didx.at[0]], add=True)
      return carry

    lax.fori_loop(0, NCHUNK, body, 0)
    plsc.subcore_barrier()
    base = c * N + s * CPS
    pltpu.sync_copy(agg_sh.at[pl.ds(base, CPW)], agg_hbm.at[pl.ds(base, CPW)])

  return agg


def _sc_edge_agg(*args):
    return _build_sc_edge_agg()(*args)


B = 1000
GRID = N // B

_HP = {"precision": lax.Precision.HIGHEST, "preferred_element_type": _f32}


def _tc1_body(x_ref, w1_ref, rw1_ref, rb1_ref, od_ref, xws_ref, res_ref):
    od = od_ref[...]
    ns = jnp.where(od > 0.0, lax.rsqrt(od), 0.0)
    x = x_ref[...]
    xws_ref[...] = jnp.dot(x, w1_ref[...], **_HP) * ns
    res_ref[...] = jnp.maximum(jnp.dot(x, rw1_ref[...], **_HP) + rb1_ref[...], 0.0)


def _tc1(x, W1, rW1, rb1, od, g):
    return pl.pallas_call(
        _tc1_body,
        grid=(GRID,),
        in_specs=[
            pl.BlockSpec((B, D), lambda i: (i, 0)),
            pl.BlockSpec((D, H), lambda i: (0, 0)),
            pl.BlockSpec((D, H), lambda i: (0, 0)),
            pl.BlockSpec((1, H), lambda i: (0, 0)),
            pl.BlockSpec((B, 1), lambda i: (i + 10 * g, 0)),
        ],
        out_specs=[pl.BlockSpec((B, H), lambda i: (i, 0)),
                   pl.BlockSpec((B, H), lambda i: (i, 0))],
        out_shape=[jax.ShapeDtypeStruct((N, H), _f32),
                   jax.ShapeDtypeStruct((N, H), _f32)],
    )(x, W1, rW1, rb1, od)


def _tc2_body(agg_ref, od_ref, id_ref, res_ref, b1_ref, g1_ref, bt1_ref,
              w2_ref, rw2_ref, rb2_ref, xws2_ref, res2_ref):
    od = od_ref[...]
    idg = id_ref[...]
    ns = jnp.where(od > 0.0, lax.rsqrt(od), 0.0)
    nd = jnp.where(idg > 0.0, lax.rsqrt(idg), 0.0)
    h = jnp.maximum(agg_ref[...] * nd + b1_ref[...], 0.0)
    h1 = g1_ref[...] * (h + res_ref[...]) + bt1_ref[...]
    xws2_ref[...] = jnp.dot(h1, w2_ref[...], **_HP) * ns
    res2_ref[...] = jnp.maximum(jnp.dot(h1, rw2_ref[...], **_HP) + rb2_ref[...], 0.0)


def _tc2(agg_all, od, idg, res1, b1, g1, bt1, W2, rW2, rb2, g):
    return pl.pallas_call(
        _tc2_body,
        grid=(GRID,),
        in_specs=[
            pl.BlockSpec((B, H), lambda i: (i + 10 * g, 0)),
            pl.BlockSpec((B, 1), lambda i: (i + 10 * g, 0)),
            pl.BlockSpec((B, 1), lambda i: (i + 10 * g, 0)),
            pl.BlockSpec((B, H), lambda i: (i, 0)),
            pl.BlockSpec((1, H), lambda i: (0, 0)),
            pl.BlockSpec((1, H), lambda i: (0, 0)),
            pl.BlockSpec((1, H), lambda i: (0, 0)),
            pl.BlockSpec((H, H), lambda i: (0, 0)),
            pl.BlockSpec((H, H), lambda i: (0, 0)),
            pl.BlockSpec((1, H), lambda i: (0, 0)),
        ],
        out_specs=[pl.BlockSpec((B, H), lambda i: (i, 0)),
                   pl.BlockSpec((B, H), lambda i: (i, 0))],
        out_shape=[jax.ShapeDtypeStruct((N, H), _f32),
                   jax.ShapeDtypeStruct((N, H), _f32)],
    )(agg_all, od, idg, res1, b1, g1, bt1, W2, rW2, rb2)


def _tc3_body(agg_ref, id_ref, res_ref, b2_ref, g2_ref, bt2_ref, aw_ref, ab_ref,
              ws_ref, mx_ref):
    idg = id_ref[...]
    nd = jnp.where(idg > 0.0, lax.rsqrt(idg), 0.0)
    h = jnp.maximum(agg_ref[...] * nd + b2_ref[...], 0.0)
    h2 = g2_ref[...] * (h + res_ref[...]) + bt2_ref[...]
    wv = jax.nn.sigmoid(jnp.dot(h2, aw_ref[...], **_HP) + ab_ref[...])
    psum = jnp.sum(wv * h2, axis=0, keepdims=True)
    pmax = jnp.max(h2, axis=0, keepdims=True)
    i = pl.program_id(0)

    @pl.when(i == 0)
    def _():
        ws_ref[...] = psum
        mx_ref[...] = pmax

    @pl.when(i > 0)
    def _():
        ws_ref[...] += psum
        mx_ref[...] = jnp.maximum(mx_ref[...], pmax)


def _tc3(agg_all, idg, res2, b2, g2, bt2, aw, ab, g):
    return pl.pallas_call(
        _tc3_body,
        grid=(GRID,),
        in_specs=[
            pl.BlockSpec((B, H), lambda i: (i + 10 * g, 0)),
            pl.BlockSpec((B, 1), lambda i: (i + 10 * g, 0)),
            pl.BlockSpec((B, H), lambda i: (i, 0)),
            pl.BlockSpec((1, H), lambda i: (0, 0)),
            pl.BlockSpec((1, H), lambda i: (0, 0)),
            pl.BlockSpec((1, H), lambda i: (0, 0)),
            pl.BlockSpec((H, 1), lambda i: (0, 0)),
            pl.BlockSpec((1, 1), lambda i: (0, 0)),
        ],
        out_specs=[pl.BlockSpec((1, H), lambda i: (0, 0)),
                   pl.BlockSpec((1, H), lambda i: (0, 0))],
        out_shape=[jax.ShapeDtypeStruct((1, H), _f32),
                   jax.ShapeDtypeStruct((1, H), _f32)],
    )(agg_all, idg, res2, b2, g2, bt2, aw, ab)


def _tc4_body(ws1, mx1, ws2, mx2,
              p1s, p1m, p1b, p1g, p1bt, p1w2, p1b2,
              p2s, p2m, p2b, p2g, p2bt, p2w2, p2b2,
              pwa, pwb, pb, out_ref):
    z1 = jnp.maximum(jnp.dot(ws1[...], p1s[...], **_HP)
                     + jnp.dot(mx1[...], p1m[...], **_HP) + p1b[...], 0.0)
    z1 = p1g[...] * z1 + p1bt[...]
    t1 = jnp.dot(z1, p1w2[...], **_HP) + p1b2[...]
    z2 = jnp.maximum(jnp.dot(ws2[...], p2s[...], **_HP)
                     + jnp.dot(mx2[...], p2m[...], **_HP) + p2b[...], 0.0)
    z2 = p2g[...] * z2 + p2bt[...]
    t2 = jnp.dot(z2, p2w2[...], **_HP) + p2b2[...]
    out_ref[...] = (jnp.dot(t1, pwa[...], **_HP)
                    + jnp.dot(t2, pwb[...], **_HP) + pb[...])


def _tc4(args):
    return pl.pallas_call(
        _tc4_body,
        out_shape=jax.ShapeDtypeStruct((1, 1), _f32),
    )(*args)


def kernel(x1, x2, edge_index1, edge_index2,
           n1_W1, n1_b1, n1_rW1, n1_rb1, n1_g1, n1_bt1,
           n1_W2, n1_b2, n1_rW2, n1_rb2, n1_g2, n1_bt2,
           n1_aw, n1_ab, n1_pW1, n1_pb1, n1_pg, n1_pbt, n1_pW2, n1_pb2,
           n2_W1, n2_b1, n2_rW1, n2_rb1, n2_g1, n2_bt1,
           n2_W2, n2_b2, n2_rW2, n2_rb2, n2_g2, n2_bt2,
           n2_aw, n2_ab, n2_pW1, n2_pb1, n2_pg, n2_pbt, n2_pW2, n2_pb2,
           predW, predb):
    srcs = jnp.concatenate([edge_index1[0], edge_index2[0] + N]).astype(jnp.int32)
    dsts = jnp.concatenate([edge_index1[1], edge_index2[1] + N]).astype(jnp.int32)
    src2d = jnp.pad(srcs.reshape(2 * NS, EPT),
                    ((0, 0), (0, EPT_PAD - EPT)),
                    constant_values=DUMP).reshape(NQ, 1, CH)
    dst2d = jnp.pad(dsts.reshape(2 * NS, EPT),
                    ((0, 0), (0, EPT_PAD - EPT)),
                    constant_values=DUMP).reshape(NQ, 1, CH)
    ones8 = jnp.ones((CH, DW), _f32)
    zeros8 = jnp.zeros((TBL_R, DW), _f32)
    zeros32 = jnp.zeros((TBL_R, H), _f32)
    pad_rows = jnp.zeros((8, H), _f32)

    od8, id8 = _sc_degrees(src2d, dst2d, ones8, zeros8)

    r = lambda v: v.reshape(1, -1)

    xws1_1, res1_1 = _tc1(x1, n1_W1, n1_rW1, r(n1_rb1), od8[:, 0:1], 0)
    xws1_2, res1_2 = _tc1(x2, n2_W1, n2_rW1, r(n2_rb1), od8[:, 0:1], 1)

    table1 = jnp.concatenate([xws1_1, xws1_2, pad_rows], axis=0)
    agg1 = _sc_edge_agg(src2d, dst2d, table1, zeros32)

    xws2_1, res2_1 = _tc2(agg1, od8[:, 0:1], id8[:, 0:1], res1_1,
                          r(n1_b1), r(n1_g1), r(n1_bt1), n1_W2, n1_rW2,
                          r(n1_rb2), 0)
    xws2_2, res2_2 = _tc2(agg1, od8[:, 0:1], id8[:, 0:1], res1_2,
                          r(n2_b1), r(n2_g1), r(n2_bt1), n2_W2, n2_rW2,
                          r(n2_rb2), 1)

    table2 = jnp.concatenate([xws2_1, xws2_2, pad_rows], axis=0)
    agg2 = _sc_edge_agg(src2d, dst2d, table2, zeros32)

    ws1, mx1 = _tc3(agg2, id8[:, 0:1], res2_1, r(n1_b2), r(n1_g2), r(n1_bt2),
                    n1_aw, n1_ab.reshape(1, 1), 0)
    ws2, mx2 = _tc3(agg2, id8[:, 0:1], res2_2, r(n2_b2), r(n2_g2), r(n2_bt2),
                    n2_aw, n2_ab.reshape(1, 1), 1)

    out = _tc4([
        ws1, mx1, ws2, mx2,
        n1_pW1[:H], n1_pW1[H:], r(n1_pb1), r(n1_pg), r(n1_pbt), n1_pW2, r(n1_pb2),
        n2_pW1[:H], n2_pW1[H:], r(n2_pb1), r(n2_pg), r(n2_pbt), n2_pW2, r(n2_pb2),
        predW[:6], predW[6:], predb.reshape(1, 1),
    ])
    return out


# speedup vs baseline: 8.5766x; 2.6914x over previous
import functools

import jax
import jax.numpy as jnp
from jax import lax
from jax.experimental import pallas as pl
from jax.experimental.pallas import tpu as pltpu
from jax.experimental.pallas import tpu_sc as plsc

N = 10000
E = 320000
D = 128
H = 32

NC = 2
NS = 16
CH = 128
EPT = E // NS
DUMP = 2 * N
TBL_R = 20224
ZCH = TBL_R // NS
CPS = 624
CPW = 640
DW = 8
VC = 160
NG = VC // 4

_f32 = jnp.float32

def _mesh():
    return plsc.VectorSubcoreMesh(
        core_axis_name="c", subcore_axis_name="s", num_cores=NC, num_subcores=NS)


@functools.lru_cache(maxsize=None)
def _build_sc_degrees():
  @functools.partial(
      pl.kernel,
      out_type=[jax.ShapeDtypeStruct((2 * N, DW), _f32),
                jax.ShapeDtypeStruct((2 * N, DW), _f32)],
      mesh=_mesh(),
      scratch_types=[
          pltpu.VMEM((VC, CH), jnp.int32),
          pltpu.VMEM((VC, CH), jnp.int32),
          pltpu.VMEM((CH, DW), _f32),
          pltpu.VMEM_SHARED((TBL_R, DW), _f32),
          pltpu.VMEM_SHARED((TBL_R, DW), _f32),
      ] + [pltpu.SemaphoreType.DMA] * 2,
      compiler_params=pltpu.CompilerParams(use_tc_tiling_on_sc=False),
  )
  def deg(src_hbm, dst_hbm, ones_hbm, zeros_hbm, od_hbm, id_hbm,
          sidx, didx, ones_v, od_sh, id_sh, *sems):
    osem = sems
    c = lax.axis_index("c")
    s = lax.axis_index("s")
    w = c * NS + s
    pltpu.sync_copy(zeros_hbm.at[pl.ds(s * ZCH, ZCH)], od_sh.at[pl.ds(s * ZCH, ZCH)])
    pltpu.sync_copy(zeros_hbm.at[pl.ds(s * ZCH, ZCH)], id_sh.at[pl.ds(s * ZCH, ZCH)])
    pltpu.sync_copy(ones_hbm, ones_v)
    pltpu.sync_copy(src_hbm.at[w], sidx)
    pltpu.sync_copy(dst_hbm.at[w], didx)
    plsc.subcore_barrier()

    def body(q, carry):
      d1 = pltpu.async_copy(ones_v, od_sh.at[sidx.at[q]], osem[0], add=True)
      d2 = pltpu.async_copy(ones_v, id_sh.at[didx.at[q]], osem[1], add=True)
      d1.wait()
      d2.wait()
      return carry

    lax.fori_loop(0, VC, body, 0)
    plsc.subcore_barrier()
    base = c * N + s * CPS
    pltpu.sync_copy(od_sh.at[pl.ds(base, CPW)], od_hbm.at[pl.ds(base, CPW)])
    pltpu.sync_copy(id_sh.at[pl.ds(base, CPW)], id_hbm.at[pl.ds(base, CPW)])

  return deg


def _sc_degrees(*args):
    return _build_sc_degrees()(*args)


@functools.lru_cache(maxsize=None)
def _build_sc_edge_agg():
  @functools.partial(
      pl.kernel,
      out_type=jax.ShapeDtypeStruct((2 * N, H), _f32),
      mesh=_mesh(),
      scratch_types=[
          pltpu.VMEM((VC, CH), jnp.int32),
          pltpu.VMEM((VC, CH), jnp.int32),
          pltpu.VMEM((CH, H), _f32),
          pltpu.VMEM((CH, H), _f32),
          pltpu.VMEM((CH, H), _f32),
          pltpu.VMEM((CH, H), _f32),
          pltpu.VMEM_SHARED((TBL_R, H), _f32),
      ] + [pltpu.SemaphoreType.DMA] * 4,
      compiler_params=pltpu.CompilerParams(use_tc_tiling_on_sc=False),
  )
  def agg(src_hbm, dst_hbm, table_hbm, zeros_hbm, agg_hbm,
          sidx, didx, r0, r1, r2, r3, agg_sh, *sems):
    rows = (r0, r1, r2, r3)
    gsem = sems
    c = lax.axis_index("c")
    s = lax.axis_index("s")
    w = c * NS + s
    pltpu.sync_copy(src_hbm.at[w], sidx)
    pltpu.sync_copy(dst_hbm.at[w], didx)
    pltpu.sync_copy(zeros_hbm.at[pl.ds(s * ZCH, ZCH)], agg_sh.at[pl.ds(s * ZCH, ZCH)])
    plsc.subcore_barrier()

    pltpu.async_copy(table_hbm.at[sidx.at[0]], rows[0], gsem[0])
    pltpu.async_copy(table_hbm.at[sidx.at[1]], rows[1], gsem[1])

    def grp(j4, carry):
      for b in range(4):
        q = 4 * j4 + b
        bp = (b + 2) % 4
        if b < 2:
          pltpu.async_copy(table_hbm.at[sidx.at[q + 2]], rows[bp], gsem[bp])
        else:
          @pl.when(j4 < NG - 1)
          def _(b=b, q=q, bp=bp):
            pltpu.async_copy(table_hbm.at[sidx.at[q + 2]], rows[bp], gsem[bp])

        pltpu.make_async_copy(table_hbm.at[sidx.at[q]], rows[b], gsem[b]).wait()
        pltpu.sync_copy(rows[b], agg_sh.at[didx.at[q]], add=True)
      return carry

    lax.fori_loop(0, NG, grp, 0)
    plsc.subcore_barrier()
    base = c * N + s * CPS
    pltpu.sync_copy(agg_sh.at[pl.ds(base, CPW)], agg_hbm.at[pl.ds(base, CPW)])

  return agg


def _sc_edge_agg(*args):
    return _build_sc_edge_agg()(*args)


B = 1000
GRID = N // B

_HP = {"precision": lax.Precision.HIGHEST, "preferred_element_type": _f32}


def _tc1_body(x_ref, w1_ref, rw1_ref, rb1_ref, od_ref, xws_ref, res_ref):
    od = od_ref[...]
    ns = jnp.where(od > 0.0, lax.rsqrt(od), 0.0)
    x = x_ref[...]
    xws_ref[...] = jnp.dot(x, w1_ref[...], **_HP) * ns
    res_ref[...] = jnp.maximum(jnp.dot(x, rw1_ref[...], **_HP) + rb1_ref[...], 0.0)


def _tc1(x, W1, rW1, rb1, od, g):
    return pl.pallas_call(
        _tc1_body,
        grid=(GRID,),
        in_specs=[
            pl.BlockSpec((B, D), lambda i: (i, 0)),
            pl.BlockSpec((D, H), lambda i: (0, 0)),
            pl.BlockSpec((D, H), lambda i: (0, 0)),
            pl.BlockSpec((1, H), lambda i: (0, 0)),
            pl.BlockSpec((B, 1), lambda i: (i + 10 * g, 0)),
        ],
        out_specs=[pl.BlockSpec((B, H), lambda i: (i, 0)),
                   pl.BlockSpec((B, H), lambda i: (i, 0))],
        out_shape=[jax.ShapeDtypeStruct((N, H), _f32),
                   jax.ShapeDtypeStruct((N, H), _f32)],
    )(x, W1, rW1, rb1, od)


def _tc2_body(agg_ref, od_ref, id_ref, res_ref, b1_ref, g1_ref, bt1_ref,
              w2_ref, rw2_ref, rb2_ref, xws2_ref, res2_ref):
    od = od_ref[...]
    idg = id_ref[...]
    ns = jnp.where(od > 0.0, lax.rsqrt(od), 0.0)
    nd = jnp.where(idg > 0.0, lax.rsqrt(idg), 0.0)
    h = jnp.maximum(agg_ref[...] * nd + b1_ref[...], 0.0)
    h1 = g1_ref[...] * (h + res_ref[...]) + bt1_ref[...]
    xws2_ref[...] = jnp.dot(h1, w2_ref[...], **_HP) * ns
    res2_ref[...] = jnp.maximum(jnp.dot(h1, rw2_ref[...], **_HP) + rb2_ref[...], 0.0)


def _tc2(agg_all, od, idg, res1, b1, g1, bt1, W2, rW2, rb2, g):
    return pl.pallas_call(
        _tc2_body,
        grid=(GRID,),
        in_specs=[
            pl.BlockSpec((B, H), lambda i: (i + 10 * g, 0)),
            pl.BlockSpec((B, 1), lambda i: (i + 10 * g, 0)),
            pl.BlockSpec((B, 1), lambda i: (i + 10 * g, 0)),
            pl.BlockSpec((B, H), lambda i: (i, 0)),
            pl.BlockSpec((1, H), lambda i: (0, 0)),
            pl.BlockSpec((1, H), lambda i: (0, 0)),
            pl.BlockSpec((1, H), lambda i: (0, 0)),
            pl.BlockSpec((H, H), lambda i: (0, 0)),
            pl.BlockSpec((H, H), lambda i: (0, 0)),
            pl.BlockSpec((1, H), lambda i: (0, 0)),
        ],
        out_specs=[pl.BlockSpec((B, H), lambda i: (i, 0)),
                   pl.BlockSpec((B, H), lambda i: (i, 0))],
        out_shape=[jax.ShapeDtypeStruct((N, H), _f32),
                   jax.ShapeDtypeStruct((N, H), _f32)],
    )(agg_all, od, idg, res1, b1, g1, bt1, W2, rW2, rb2)


def _tc3_body(agg_ref, id_ref, res_ref, b2_ref, g2_ref, bt2_ref, aw_ref, ab_ref,
              ws_ref, mx_ref):
    idg = id_ref[...]
    nd = jnp.where(idg > 0.0, lax.rsqrt(idg), 0.0)
    h = jnp.maximum(agg_ref[...] * nd + b2_ref[...], 0.0)
    h2 = g2_ref[...] * (h + res_ref[...]) + bt2_ref[...]
    wv = jax.nn.sigmoid(jnp.dot(h2, aw_ref[...], **_HP) + ab_ref[...])
    psum = jnp.sum(wv * h2, axis=0, keepdims=True)
    pmax = jnp.max(h2, axis=0, keepdims=True)
    i = pl.program_id(0)

    @pl.when(i == 0)
    def _():
        ws_ref[...] = psum
        mx_ref[...] = pmax

    @pl.when(i > 0)
    def _():
        ws_ref[...] += psum
        mx_ref[...] = jnp.maximum(mx_ref[...], pmax)


def _tc3(agg_all, idg, res2, b2, g2, bt2, aw, ab, g):
    return pl.pallas_call(
        _tc3_body,
        grid=(GRID,),
        in_specs=[
            pl.BlockSpec((B, H), lambda i: (i + 10 * g, 0)),
            pl.BlockSpec((B, 1), lambda i: (i + 10 * g, 0)),
            pl.BlockSpec((B, H), lambda i: (i, 0)),
            pl.BlockSpec((1, H), lambda i: (0, 0)),
            pl.BlockSpec((1, H), lambda i: (0, 0)),
            pl.BlockSpec((1, H), lambda i: (0, 0)),
            pl.BlockSpec((H, 1), lambda i: (0, 0)),
            pl.BlockSpec((1, 1), lambda i: (0, 0)),
        ],
        out_specs=[pl.BlockSpec((1, H), lambda i: (0, 0)),
                   pl.BlockSpec((1, H), lambda i: (0, 0))],
        out_shape=[jax.ShapeDtypeStruct((1, H), _f32),
                   jax.ShapeDtypeStruct((1, H), _f32)],
    )(agg_all, idg, res2, b2, g2, bt2, aw, ab)


def _tc4_body(ws1, mx1, ws2, mx2,
              p1s, p1m, p1b, p1g, p1bt, p1w2, p1b2,
              p2s, p2m, p2b, p2g, p2bt, p2w2, p2b2,
              pwa, pwb, pb, out_ref):
    z1 = jnp.maximum(jnp.dot(ws1[...], p1s[...], **_HP)
                     + jnp.dot(mx1[...], p1m[...], **_HP) + p1b[...], 0.0)
    z1 = p1g[...] * z1 + p1bt[...]
    t1 = jnp.dot(z1, p1w2[...], **_HP) + p1b2[...]
    z2 = jnp.maximum(jnp.dot(ws2[...], p2s[...], **_HP)
                     + jnp.dot(mx2[...], p2m[...], **_HP) + p2b[...], 0.0)
    z2 = p2g[...] * z2 + p2bt[...]
    t2 = jnp.dot(z2, p2w2[...], **_HP) + p2b2[...]
    out_ref[...] = (jnp.dot(t1, pwa[...], **_HP)
                    + jnp.dot(t2, pwb[...], **_HP) + pb[...])


def _tc4(args):
    return pl.pallas_call(
        _tc4_body,
        out_shape=jax.ShapeDtypeStruct((1, 1), _f32),
    )(*args)


def kernel(x1, x2, edge_index1, edge_index2,
           n1_W1, n1_b1, n1_rW1, n1_rb1, n1_g1, n1_bt1,
           n1_W2, n1_b2, n1_rW2, n1_rb2, n1_g2, n1_bt2,
           n1_aw, n1_ab, n1_pW1, n1_pb1, n1_pg, n1_pbt, n1_pW2, n1_pb2,
           n2_W1, n2_b1, n2_rW1, n2_rb1, n2_g1, n2_bt1,
           n2_W2, n2_b2, n2_rW2, n2_rb2, n2_g2, n2_bt2,
           n2_aw, n2_ab, n2_pW1, n2_pb1, n2_pg, n2_pbt, n2_pW2, n2_pb2,
           predW, predb):
    srcs = jnp.concatenate([edge_index1[0], edge_index2[0] + N]).astype(jnp.int32)
    dsts = jnp.concatenate([edge_index1[1], edge_index2[1] + N]).astype(jnp.int32)
    padv = (DUMP + (jnp.arange(2 * NS, dtype=jnp.int32) % NS))[:, None]
    pad_blk = jnp.broadcast_to(padv, (2 * NS, VC * CH - EPT))
    src2d = jnp.concatenate([srcs.reshape(2 * NS, EPT), pad_blk],
                            axis=1).reshape(2 * NS, VC, CH)
    dst2d = jnp.concatenate([dsts.reshape(2 * NS, EPT), pad_blk],
                            axis=1).reshape(2 * NS, VC, CH)
    ones8 = jnp.ones((CH, DW), _f32)
    zeros8 = jnp.zeros((TBL_R, DW), _f32)
    zeros32 = jnp.zeros((TBL_R, H), _f32)
    pad_rows = jnp.zeros((NS, H), _f32)

    od8, id8 = _sc_degrees(src2d, dst2d, ones8, zeros8)

    r = lambda v: v.reshape(1, -1)

    xws1_1, res1_1 = _tc1(x1, n1_W1, n1_rW1, r(n1_rb1), od8[:, 0:1], 0)
    xws1_2, res1_2 = _tc1(x2, n2_W1, n2_rW1, r(n2_rb1), od8[:, 0:1], 1)

    table1 = jnp.concatenate([xws1_1, xws1_2, pad_rows], axis=0)
    agg1 = _sc_edge_agg(src2d, dst2d, table1, zeros32)

    xws2_1, res2_1 = _tc2(agg1, od8[:, 0:1], id8[:, 0:1], res1_1,
                          r(n1_b1), r(n1_g1), r(n1_bt1), n1_W2, n1_rW2,
                          r(n1_rb2), 0)
    xws2_2, res2_2 = _tc2(agg1, od8[:, 0:1], id8[:, 0:1], res1_2,
                          r(n2_b1), r(n2_g1), r(n2_bt1), n2_W2, n2_rW2,
                          r(n2_rb2), 1)

    table2 = jnp.concatenate([xws2_1, xws2_2, pad_rows], axis=0)
    agg2 = _sc_edge_agg(src2d, dst2d, table2, zeros32)

    ws1, mx1 = _tc3(agg2, id8[:, 0:1], res2_1, r(n1_b2), r(n1_g2), r(n1_bt2),
                    n1_aw, n1_ab.reshape(1, 1), 0)
    ws2, mx2 = _tc3(agg2, id8[:, 0:1], res2_2, r(n2_b2), r(n2_g2), r(n2_bt2),
                    n2_aw, n2_ab.reshape(1, 1), 1)

    out = _tc4([
        ws1, mx1, ws2, mx2,
        n1_pW1[:H], n1_pW1[H:], r(n1_pb1), r(n1_pg), r(n1_pbt), n1_pW2, r(n1_pb2),
        n2_pW1[:H], n2_pW1[H:], r(n2_pb1), r(n2_pg), r(n2_pbt), n2_pW2, r(n2_pb2),
        predW[:6], predW[6:], predb.reshape(1, 1),
    ])
    return out
